# baseline (speedup 1.0000x reference)
import functools

import jax
import jax.numpy as jnp
from jax import lax
from jax.experimental import pallas as pl
from jax.experimental.pallas import tpu as pltpu
from jax.experimental.pallas import tpu_sc as plsc

_NUM_CORES = 2
_NUM_SUBCORES = 16
_NW = _NUM_CORES * _NUM_SUBCORES

_SB = 8
_BB = 128


def _gather_t(table2, idx_t, n_s, d, n_b):
    assert n_b // _BB == _NW
    w = 2 * d
    mesh = plsc.VectorSubcoreMesh(core_axis_name="c", subcore_axis_name="s")

    @functools.partial(
        pl.kernel,
        mesh=mesh,
        out_type=jax.ShapeDtypeStruct((n_s, d, n_b), jnp.float32),
        compiler_params=pltpu.CompilerParams(needs_layout_passes=False),
        scratch_types=[
            pltpu.VMEM((_SB, _BB), jnp.int32),
            pltpu.VMEM((2, _BB), jnp.int32),
            pltpu.VMEM((2, _BB), jnp.int32),
            pltpu.VMEM((2, _BB, 2 * d), jnp.float32),
            pltpu.VMEM((2, d, _BB + 1), jnp.float32),
            pltpu.SemaphoreType.DMA((2,)),
            pltpu.SemaphoreType.DMA((2,)),
        ],
    )
    def g_kernel(table_hbm, idx_hbm, out_hbm, idxt_v, pv, base_v, pair_v,
                 outb_v, sem_g, sem_wb):
        wid = lax.axis_index("s") * _NUM_CORES + lax.axis_index("c")
        b0 = wid * _BB
        lane = lax.iota(jnp.int32, 16)

        def fire(t):
            slot = lax.rem(t, 2)
            si = lax.rem(t, _SB)

            @pl.when(si == 0)
            def _stage():
                ts = pl.multiple_of(t, _SB)
                pltpu.sync_copy(
                    idx_hbm.at[pl.ds(ts, _SB), pl.ds(b0, _BB)], idxt_v)

            for jb in range(_BB // 16):
                vv = idxt_v[si, pl.ds(jb * 16, 16)]
                pv[slot, pl.ds(jb * 16, 16)] = lax.shift_right_logical(vv, 1)
                base_v[slot, pl.ds(jb * 16, 16)] = (vv & 1) * d
            pltpu.async_copy(table_hbm.at[pv.at[slot]], pair_v.at[slot],
                             sem_g.at[slot])

        def transpose(t):
            slot = lax.rem(t, 2)

            def do_j4(jq, c3):
                for u in range(4):
                    j = jq * 4 + u
                    js = jnp.full((16,), 0, jnp.int32) + j
                    hs = plsc.load_gather(base_v.at[slot], [js])
                    for d0 in range(0, d, 16):
                        x = plsc.load_gather(pair_v.at[slot],
                                             [js, hs + (lane + d0)])
                        plsc.store_scatter(outb_v.at[slot],
                                           [lane + d0, js], x)
                return c3

            lax.fori_loop(0, _BB // 4, do_j4, 0)

        fire(0)

        def body(t, carry):
            slot = lax.rem(t, 2)

            @pl.when(t < n_s - 1)
            def _prefetch():
                fire(t + 1)

            pltpu.make_async_copy(table_hbm.at[pl.ds(0, _BB)],
                                  pair_v.at[slot], sem_g.at[slot]).wait()

            @pl.when(t >= 2)
            def _drain_wb():
                pltpu.make_async_copy(outb_v.at[slot, :, pl.ds(0, _BB)],
                                      out_hbm.at[0, :, pl.ds(b0, _BB)],
                                      sem_wb.at[slot]).wait()

            transpose(t)
            pltpu.async_copy(outb_v.at[slot, :, pl.ds(0, _BB)],
                             out_hbm.at[t, :, pl.ds(b0, _BB)],
                             sem_wb.at[slot])
            return carry

        lax.fori_loop(0, n_s, body, 0)
        for slot in range(2):
            pltpu.make_async_copy(outb_v.at[slot, :, pl.ds(0, _BB)],
                                  out_hbm.at[0, :, pl.ds(b0, _BB)],
                                  sem_wb.at[slot]).wait()

    return g_kernel(table2, idx_t)


def kernel(position_labels, pos_embedding_weight):
    b, s = position_labels.shape
    v, d = pos_embedding_weight.shape
    idx_t = position_labels.T.astype(jnp.int32)
    table2 = pos_embedding_weight.reshape(v // 2, 2 * d)
    out_t = _gather_t(table2, idx_t, s, d, b)
    return out_t.transpose(2, 0, 1)


# speedup vs baseline: 1.5572x; 1.5572x over previous
import functools

import jax
import jax.numpy as jnp
from jax import lax
from jax.experimental import pallas as pl
from jax.experimental.pallas import tpu as pltpu
from jax.experimental.pallas import tpu_sc as plsc

_NUM_CORES = 2
_NUM_SUBCORES = 16
_NW = _NUM_CORES * _NUM_SUBCORES

_SB = 8
_BB = 128


def _gather_t(table2, idx_t, n_s, d, n_b):
    assert n_b // _BB == _NW
    w = 2 * d
    mesh = plsc.VectorSubcoreMesh(core_axis_name="c", subcore_axis_name="s")

    @functools.partial(
        pl.kernel,
        mesh=mesh,
        out_type=jax.ShapeDtypeStruct((n_s, d, n_b), jnp.float32),
        compiler_params=pltpu.CompilerParams(needs_layout_passes=False),
        scratch_types=[
            pltpu.VMEM((_SB, _BB), jnp.int32),
            pltpu.VMEM((2, _BB), jnp.int32),
            pltpu.VMEM((2, _BB), jnp.int32),
            pltpu.VMEM((2, _BB, 3 * d), jnp.float32),
            pltpu.VMEM((2, d, _BB), jnp.float32),
            pltpu.SemaphoreType.DMA((2,)),
            pltpu.SemaphoreType.DMA((2,)),
        ],
    )
    def g_kernel(table_hbm, idx_hbm, out_hbm, idxt_v, pv, base_v, pair_v,
                 outb_v, sem_g, sem_wb):
        wid = lax.axis_index("s") * _NUM_CORES + lax.axis_index("c")
        b0 = wid * _BB
        lane = lax.iota(jnp.int32, 16)

        def fire(t):
            slot = lax.rem(t, 2)
            si = lax.rem(t, _SB)

            @pl.when(si == 0)
            def _stage():
                ts = pl.multiple_of(t, _SB)
                pltpu.sync_copy(
                    idx_hbm.at[pl.ds(ts, _SB), pl.ds(b0, _BB)], idxt_v)

            for jb in range(_BB // 16):
                vv = idxt_v[si, pl.ds(jb * 16, 16)]
                pv[slot, pl.ds(jb * 16, 16)] = lax.shift_right_logical(vv, 1)
                base_v[slot, pl.ds(jb * 16, 16)] = (vv & 1) * d
            pltpu.async_copy(table_hbm.at[pv.at[slot]],
                             pair_v.at[slot, :, pl.ds(0, 2 * d)],
                             sem_g.at[slot])

        def transpose(t):
            slot = lax.rem(t, 2)
            for jb in range(_BB // 16):
                vb = base_v[slot, pl.ds(jb * 16, 16)]
                rj = lane + jb * 16
                for d0 in range(0, d, 8):
                    xs = [
                        plsc.load_gather(pair_v.at[slot], [rj, vb + d0 + i])
                        for i in range(8)
                    ]
                    for i in range(8):
                        outb_v[slot, d0 + i, pl.ds(jb * 16, 16)] = xs[i]

        fire(0)

        def body(t, carry):
            slot = lax.rem(t, 2)

            @pl.when(t < n_s - 1)
            def _prefetch():
                fire(t + 1)

            pltpu.make_async_copy(table_hbm.at[pl.ds(0, _BB)],
                                  pair_v.at[slot, :, pl.ds(0, 2 * d)],
                                  sem_g.at[slot]).wait()

            @pl.when(t >= 2)
            def _drain_wb():
                pltpu.make_async_copy(outb_v.at[slot],
                                      out_hbm.at[0, :, pl.ds(b0, _BB)],
                                      sem_wb.at[slot]).wait()

            transpose(t)
            pltpu.async_copy(outb_v.at[slot],
                             out_hbm.at[t, :, pl.ds(b0, _BB)],
                             sem_wb.at[slot])
            return carry

        lax.fori_loop(0, n_s, body, 0)
        for slot in range(2):
            pltpu.make_async_copy(outb_v.at[slot],
                                  out_hbm.at[0, :, pl.ds(b0, _BB)],
                                  sem_wb.at[slot]).wait()

    return g_kernel(table2, idx_t)


def kernel(position_labels, pos_embedding_weight):
    b, s = position_labels.shape
    v, d = pos_embedding_weight.shape
    idx_t = position_labels.T.astype(jnp.int32)
    table2 = pos_embedding_weight.reshape(v // 2, 2 * d)
    out_t = _gather_t(table2, idx_t, s, d, b)
    return out_t.transpose(2, 0, 1)
